# baseline (speedup 1.0000x reference)
import functools

import jax
import jax.numpy as jnp
from jax import lax
from jax.experimental import pallas as pl
from jax.experimental.pallas import tpu as pltpu
from jax.experimental.pallas import tpu_sc as plsc

_B = 4096
_D = 2048
_C = 64
_OUT = 4096
_COUNT = 2048

_NC = 2
_NS = 16
_NW = _NC * _NS
_ROWS_PER_TILE = _B // _NW
_RCH = 8
_CHUNKS = _ROWS_PER_TILE // _RCH
_L = 16


def _coef_body(lg_ref, clg_ref, co_ref, dc_ref):
    w = 0.5 + 0.5 * jnp.sin(lg_ref[...])
    cw = 0.5 + 0.5 * jnp.sin(clg_ref[...])

    def poly(t):
        w00 = t[0:1, :]
        w01 = t[1:2, :]
        w10 = t[2:3, :]
        w11 = t[3:4, :]
        return jnp.concatenate(
            [w00, w10 - w00, w01 - w00, w11 - w10 - w01 + w00], axis=0)

    co = poly(w)
    cc = poly(cw)
    co_ref[...] = co
    dc_ref[...] = cc - co[:, :_COUNT]


def _coefs(logits, code_logits):
    return pl.pallas_call(
        _coef_body,
        out_shape=[
            jax.ShapeDtypeStruct((4, _OUT), jnp.float32),
            jax.ShapeDtypeStruct((4, _COUNT), jnp.float32),
        ],
    )(logits.T, code_logits.T)


_ROW = _D + _C


def _sc_body(x_hbm, lidx_hbm, ridx_hbm, cidx_hbm, co_hbm, dc_hbm, out_hbm,
             lidx_v, ridx_v, cidx_v, co_v, dc_v, xch, och0, och1,
             osem0, osem1):
    wid = lax.axis_index("s") * _NC + lax.axis_index("c")

    pltpu.sync_copy(lidx_hbm, lidx_v)
    pltpu.sync_copy(ridx_hbm, ridx_v)
    pltpu.sync_copy(cidx_hbm, cidx_v)
    pltpu.sync_copy(co_hbm, co_v)
    pltpu.sync_copy(dc_hbm, dc_v)

    row0 = wid * _ROWS_PER_TILE
    ochs = (och0, och1)
    osems = (osem0, osem1)

    def in_slice(ch):
        return x_hbm.at[pl.ds((row0 + ch * _RCH) * _ROW, _RCH * _ROW)]

    def out_slice(ch):
        return out_hbm.at[pl.ds(row0 + ch * _RCH, _RCH)]

    def compute(xch, och):
        @plsc.parallel_loop(0, _COUNT // 128)
        def blended(jt):
            jb = jt * 128
            for jr in range(128 // _L):
                jo = jb + jr * _L
                li = lidx_v[pl.ds(jo, _L)]
                ri = ridx_v[pl.ds(jo, _L)]
                ci = cidx_v[pl.ds(jo, _L)]
                a = co_v[pl.ds(jo, _L)]
                bl = co_v[pl.ds(_OUT + jo, _L)]
                br = co_v[pl.ds(2 * _OUT + jo, _L)]
                bb = co_v[pl.ds(3 * _OUT + jo, _L)]
                da = dc_v[pl.ds(jo, _L)]
                dbl = dc_v[pl.ds(_COUNT + jo, _L)]
                dbr = dc_v[pl.ds(2 * _COUNT + jo, _L)]
                dbb = dc_v[pl.ds(3 * _COUNT + jo, _L)]
                for i in range(_RCH):
                    l = plsc.load_gather(xch, [li + i * _ROW])
                    r = plsc.load_gather(xch, [ri + i * _ROW])
                    s = plsc.load_gather(xch, [ci + (i * _ROW + _D)])
                    lr = l * r
                    p = a + bl * l + br * r + bb * lr
                    dp = da + dbl * l + dbr * r + dbb * lr
                    och[i, pl.ds(jo, _L)] = p + s * dp

        @plsc.parallel_loop(_COUNT // 128, _OUT // 128)
        def plain(jt):
            jb = jt * 128
            for jr in range(128 // _L):
                jo = jb + jr * _L
                li = lidx_v[pl.ds(jo, _L)]
                ri = ridx_v[pl.ds(jo, _L)]
                a = co_v[pl.ds(jo, _L)]
                bl = co_v[pl.ds(_OUT + jo, _L)]
                br = co_v[pl.ds(2 * _OUT + jo, _L)]
                bb = co_v[pl.ds(3 * _OUT + jo, _L)]
                for i in range(_RCH):
                    l = plsc.load_gather(xch, [li + i * _ROW])
                    r = plsc.load_gather(xch, [ri + i * _ROW])
                    och[i, pl.ds(jo, _L)] = a + bl * l + br * r + bb * (l * r)

        del blended, plain

    def pair(p, carry):
        for b in range(2):
            ch = 2 * p + b
            pltpu.sync_copy(in_slice(ch), xch)

            @pl.when(ch >= 2)
            def _():
                pltpu.make_async_copy(ochs[b], out_slice(ch), osems[b]).wait()

            compute(xch, ochs[b])
            pltpu.async_copy(ochs[b], out_slice(ch), osems[b])
        return carry

    lax.fori_loop(0, _CHUNKS // 2, pair, 0)

    pltpu.make_async_copy(och0, out_slice(_CHUNKS - 2), osem0).wait()
    pltpu.make_async_copy(och1, out_slice(_CHUNKS - 1), osem1).wait()


_sc_call = functools.partial(
    pl.kernel,
    out_type=jax.ShapeDtypeStruct((_B, _OUT), jnp.float32),
    mesh=plsc.VectorSubcoreMesh(core_axis_name="c", subcore_axis_name="s"),
    compiler_params=pltpu.CompilerParams(needs_layout_passes=False),
    scratch_types=[
        pltpu.VMEM((_OUT,), jnp.int32),
        pltpu.VMEM((_OUT,), jnp.int32),
        pltpu.VMEM((_COUNT,), jnp.int32),
        pltpu.VMEM((4 * _OUT,), jnp.float32),
        pltpu.VMEM((4 * _COUNT,), jnp.float32),
        pltpu.VMEM((_RCH * _ROW,), jnp.float32),
        pltpu.VMEM((_RCH, _OUT), jnp.float32),
        pltpu.VMEM((_RCH, _OUT), jnp.float32),
        pltpu.SemaphoreType.DMA,
        pltpu.SemaphoreType.DMA,
    ],
)(_sc_body)


@jax.jit
def kernel(x, logits, code_logits, left_indices, right_indices, code_indices):
    co, dc = _coefs(logits, code_logits)
    return _sc_call(
        x.reshape(-1),
        left_indices.astype(jnp.int32),
        right_indices.astype(jnp.int32),
        code_indices.astype(jnp.int32),
        co.reshape(-1),
        dc.reshape(-1),
    )


# speedup vs baseline: 1.2454x; 1.2454x over previous
import functools

import jax
import jax.numpy as jnp
from jax import lax
from jax.experimental import pallas as pl
from jax.experimental.pallas import tpu as pltpu
from jax.experimental.pallas import tpu_sc as plsc

_B = 4096
_D = 2048
_C = 64
_OUT = 4096
_COUNT = 2048

_NC = 2
_NS = 16
_NW = _NC * _NS
_ROWS_PER_TILE = _B // _NW
_RCH = 8
_CHUNKS = _ROWS_PER_TILE // _RCH
_L = 16


def _coef_body(lg_ref, clg_ref, co_ref, dc_ref):
    w = 0.5 + 0.5 * jnp.sin(lg_ref[...])
    cw = 0.5 + 0.5 * jnp.sin(clg_ref[...])

    def poly(t):
        w00 = t[0:1, :]
        w01 = t[1:2, :]
        w10 = t[2:3, :]
        w11 = t[3:4, :]
        return jnp.concatenate(
            [w00, w10 - w00, w01 - w00, w11 - w10 - w01 + w00], axis=0)

    co = poly(w)
    cc = poly(cw)
    co_ref[...] = co
    dc_ref[...] = cc - co[:, :_COUNT]


def _coefs(logits, code_logits):
    return pl.pallas_call(
        _coef_body,
        out_shape=[
            jax.ShapeDtypeStruct((4, _OUT), jnp.float32),
            jax.ShapeDtypeStruct((4, _COUNT), jnp.float32),
        ],
    )(logits.T, code_logits.T)


_ROW = _D + _C


def _sc_body(x_hbm, lidx_hbm, ridx_hbm, cidx_hbm, co_hbm, dc_hbm, out_hbm,
             lidx_v, ridx_v, cidx_v, co_v, dc_v, xch0, xch1, ocha, ochb,
             isem0, isem1, osema, osemb):
    wid = lax.axis_index("s") * _NC + lax.axis_index("c")

    pltpu.sync_copy(lidx_hbm, lidx_v)
    pltpu.sync_copy(ridx_hbm, ridx_v)
    pltpu.sync_copy(cidx_hbm, cidx_v)
    pltpu.sync_copy(co_hbm, co_v)
    pltpu.sync_copy(dc_hbm, dc_v)

    row0 = wid * _ROWS_PER_TILE
    xchs = (xch0, xch1)
    isems = (isem0, isem1)

    def in_slice(ch):
        return x_hbm.at[pl.ds((row0 + ch * _RCH) * _ROW, _RCH * _ROW)]

    def outa_slice(ch):
        return out_hbm.at[pl.ds(row0 + ch * _RCH, _RCH), pl.ds(0, _COUNT)]

    def outb_slice(ch):
        return out_hbm.at[pl.ds(row0 + ch * _RCH, _RCH), pl.ds(_COUNT, _COUNT)]

    def compute_blended(xch, och):
        @plsc.parallel_loop(0, _COUNT // _L, unroll=2)
        def blended(jv):
            jo = jv * _L
            li = lidx_v[pl.ds(jo, _L)]
            ri = ridx_v[pl.ds(jo, _L)]
            ci = cidx_v[pl.ds(jo, _L)]
            a = co_v[pl.ds(jo, _L)]
            bl = co_v[pl.ds(_OUT + jo, _L)]
            br = co_v[pl.ds(2 * _OUT + jo, _L)]
            bb = co_v[pl.ds(3 * _OUT + jo, _L)]
            da = dc_v[pl.ds(jo, _L)]
            dbl = dc_v[pl.ds(_COUNT + jo, _L)]
            dbr = dc_v[pl.ds(2 * _COUNT + jo, _L)]
            dbb = dc_v[pl.ds(3 * _COUNT + jo, _L)]
            for i in range(_RCH):
                l = plsc.load_gather(xch, [li + i * _ROW])
                r = plsc.load_gather(xch, [ri + i * _ROW])
                s = plsc.load_gather(xch, [ci + (i * _ROW + _D)])
                lr = l * r
                p = a + bl * l + br * r + bb * lr
                dp = da + dbl * l + dbr * r + dbb * lr
                och[i, pl.ds(jo, _L)] = p + s * dp

        del blended

    def compute_plain(xch, och):
        @plsc.parallel_loop(_COUNT // _L, _OUT // _L, unroll=2)
        def plain(jv):
            jo = jv * _L
            li = lidx_v[pl.ds(jo, _L)]
            ri = ridx_v[pl.ds(jo, _L)]
            a = co_v[pl.ds(jo, _L)]
            bl = co_v[pl.ds(_OUT + jo, _L)]
            br = co_v[pl.ds(2 * _OUT + jo, _L)]
            bb = co_v[pl.ds(3 * _OUT + jo, _L)]
            for i in range(_RCH):
                l = plsc.load_gather(xch, [li + i * _ROW])
                r = plsc.load_gather(xch, [ri + i * _ROW])
                och[i, pl.ds(jo - _COUNT, _L)] = a + bl * l + br * r + bb * (l * r)

        del plain

    pltpu.async_copy(in_slice(0), xch0, isem0)

    def pair(p, carry):
        for b in range(2):
            ch = 2 * p + b

            @pl.when(ch + 1 < _CHUNKS)
            def _():
                pltpu.async_copy(in_slice(ch + 1), xchs[1 - b], isems[1 - b])

            pltpu.make_async_copy(in_slice(ch), xchs[b], isems[b]).wait()

            @pl.when(ch >= 1)
            def _():
                pltpu.make_async_copy(ocha, outa_slice(ch), osema).wait()

            compute_blended(xchs[b], ocha)
            pltpu.async_copy(ocha, outa_slice(ch), osema)

            @pl.when(ch >= 1)
            def _():
                pltpu.make_async_copy(ochb, outb_slice(ch), osemb).wait()

            compute_plain(xchs[b], ochb)
            pltpu.async_copy(ochb, outb_slice(ch), osemb)
        return carry

    lax.fori_loop(0, _CHUNKS // 2, pair, 0)

    pltpu.make_async_copy(ocha, outa_slice(_CHUNKS - 1), osema).wait()
    pltpu.make_async_copy(ochb, outb_slice(_CHUNKS - 1), osemb).wait()


_sc_call = functools.partial(
    pl.kernel,
    out_type=jax.ShapeDtypeStruct((_B, _OUT), jnp.float32),
    mesh=plsc.VectorSubcoreMesh(core_axis_name="c", subcore_axis_name="s"),
    compiler_params=pltpu.CompilerParams(needs_layout_passes=False),
    scratch_types=[
        pltpu.VMEM((_OUT,), jnp.int32),
        pltpu.VMEM((_OUT,), jnp.int32),
        pltpu.VMEM((_COUNT,), jnp.int32),
        pltpu.VMEM((4 * _OUT,), jnp.float32),
        pltpu.VMEM((4 * _COUNT,), jnp.float32),
        pltpu.VMEM((_RCH * _ROW,), jnp.float32),
        pltpu.VMEM((_RCH * _ROW,), jnp.float32),
        pltpu.VMEM((_RCH, _COUNT), jnp.float32),
        pltpu.VMEM((_RCH, _COUNT), jnp.float32),
        pltpu.SemaphoreType.DMA,
        pltpu.SemaphoreType.DMA,
        pltpu.SemaphoreType.DMA,
        pltpu.SemaphoreType.DMA,
    ],
)(_sc_body)


@jax.jit
def kernel(x, logits, code_logits, left_indices, right_indices, code_indices):
    co, dc = _coefs(logits, code_logits)
    return _sc_call(
        x.reshape(-1),
        left_indices.astype(jnp.int32),
        right_indices.astype(jnp.int32),
        code_indices.astype(jnp.int32),
        co.reshape(-1),
        dc.reshape(-1),
    )


# speedup vs baseline: 1.7515x; 1.4064x over previous
import functools

import jax
import jax.numpy as jnp
from jax import lax
from jax.experimental import pallas as pl
from jax.experimental.pallas import tpu as pltpu
from jax.experimental.pallas import tpu_sc as plsc

_B = 4096
_D = 2048
_C = 64
_OUT = 4096
_COUNT = 2048

_NC = 2
_NS = 16
_NW = _NC * _NS
_ROWS_PER_TILE = _B // _NW
_RCH = 8
_CHUNKS = _ROWS_PER_TILE // _RCH
_L = 16


def _coef_body(lg_ref, clg_ref, co_ref, dc_ref):
    w = 0.5 + 0.5 * jnp.sin(lg_ref[...])
    cw = 0.5 + 0.5 * jnp.sin(clg_ref[...])

    def poly(t):
        w00 = t[0:1, :]
        w01 = t[1:2, :]
        w10 = t[2:3, :]
        w11 = t[3:4, :]
        return jnp.concatenate(
            [w00, w10 - w00, w01 - w00, w11 - w10 - w01 + w00], axis=0)

    co = poly(w)
    cc = poly(cw)
    co_ref[...] = co
    dc_ref[...] = cc - co[:, :_COUNT]


def _coefs(logits, code_logits):
    return pl.pallas_call(
        _coef_body,
        out_shape=[
            jax.ShapeDtypeStruct((4, _OUT), jnp.float32),
            jax.ShapeDtypeStruct((4, _COUNT), jnp.float32),
        ],
    )(logits.T, code_logits.T)


_ROW = _D + _C


def _sc_body(x_hbm, lidx_hbm, ridx_hbm, cidx_hbm, co_hbm, dc_hbm, out_hbm,
             lidx_v, ridx_v, cidx_v, co_v, dc_v, xch0, xch1, ocha, ochb,
             isem0, isem1, osema, osemb):
    wid = lax.axis_index("s") * _NC + lax.axis_index("c")

    pltpu.sync_copy(lidx_hbm, lidx_v)
    pltpu.sync_copy(ridx_hbm, ridx_v)
    pltpu.sync_copy(cidx_hbm, cidx_v)
    pltpu.sync_copy(co_hbm, co_v)
    pltpu.sync_copy(dc_hbm, dc_v)

    row0 = wid * _ROWS_PER_TILE
    xchs = (xch0, xch1)
    isems = (isem0, isem1)

    def in_slice(ch):
        return x_hbm.at[pl.ds((row0 + ch * _RCH) * _ROW, _RCH * _ROW)]

    def outa_slice(ch):
        return out_hbm.at[pl.ds(row0 + ch * _RCH, _RCH), pl.ds(0, _COUNT)]

    def outb_slice(ch):
        return out_hbm.at[pl.ds(row0 + ch * _RCH, _RCH), pl.ds(_COUNT, _COUNT)]

    def compute_blended(xch, och):
        @plsc.parallel_loop(0, _COUNT // _L, unroll=1)
        def blended(jv):
            jo = jv * _L
            li = lidx_v[pl.ds(jo, _L)]
            ri = ridx_v[pl.ds(jo, _L)]
            ci = cidx_v[pl.ds(jo, _L)]
            a = co_v[pl.ds(jo, _L)]
            bl = co_v[pl.ds(_OUT + jo, _L)]
            br = co_v[pl.ds(2 * _OUT + jo, _L)]
            bb = co_v[pl.ds(3 * _OUT + jo, _L)]
            da = dc_v[pl.ds(jo, _L)]
            dbl = dc_v[pl.ds(_COUNT + jo, _L)]
            dbr = dc_v[pl.ds(2 * _COUNT + jo, _L)]
            dbb = dc_v[pl.ds(3 * _COUNT + jo, _L)]
            for i in range(_RCH):
                l = plsc.load_gather(xch, [li + i * _ROW])
                r = plsc.load_gather(xch, [ri + i * _ROW])
                s = plsc.load_gather(xch, [ci + (i * _ROW + _D)])
                lr = l * r
                p = a + bl * l + br * r + bb * lr
                dp = da + dbl * l + dbr * r + dbb * lr
                och[i, pl.ds(jo, _L)] = p + s * dp

        del blended

    def compute_plain(xch, och):
        @plsc.parallel_loop(_COUNT // _L, _OUT // _L, unroll=1)
        def plain(jv):
            jo = jv * _L
            li = lidx_v[pl.ds(jo, _L)]
            ri = ridx_v[pl.ds(jo, _L)]
            a = co_v[pl.ds(jo, _L)]
            bl = co_v[pl.ds(_OUT + jo, _L)]
            br = co_v[pl.ds(2 * _OUT + jo, _L)]
            bb = co_v[pl.ds(3 * _OUT + jo, _L)]
            for i in range(_RCH):
                l = plsc.load_gather(xch, [li + i * _ROW])
                r = plsc.load_gather(xch, [ri + i * _ROW])
                och[i, pl.ds(jo - _COUNT, _L)] = a + bl * l + br * r + bb * (l * r)

        del plain

    pltpu.async_copy(in_slice(0), xch0, isem0)

    def pair(p, carry):
        for b in range(2):
            ch = 2 * p + b

            @pl.when(ch + 1 < _CHUNKS)
            def _():
                pltpu.async_copy(in_slice(ch + 1), xchs[1 - b], isems[1 - b])

            pltpu.make_async_copy(in_slice(ch), xchs[b], isems[b]).wait()

            @pl.when(ch >= 1)
            def _():
                pltpu.make_async_copy(ocha, outa_slice(ch), osema).wait()

            compute_blended(xchs[b], ocha)
            pltpu.async_copy(ocha, outa_slice(ch), osema)

            @pl.when(ch >= 1)
            def _():
                pltpu.make_async_copy(ochb, outb_slice(ch), osemb).wait()

            compute_plain(xchs[b], ochb)
            pltpu.async_copy(ochb, outb_slice(ch), osemb)
        return carry

    lax.fori_loop(0, _CHUNKS // 2, pair, 0)

    pltpu.make_async_copy(ocha, outa_slice(_CHUNKS - 1), osema).wait()
    pltpu.make_async_copy(ochb, outb_slice(_CHUNKS - 1), osemb).wait()


_sc_call = functools.partial(
    pl.kernel,
    out_type=jax.ShapeDtypeStruct((_B, _OUT), jnp.float32),
    mesh=plsc.VectorSubcoreMesh(core_axis_name="c", subcore_axis_name="s"),
    compiler_params=pltpu.CompilerParams(needs_layout_passes=False),
    scratch_types=[
        pltpu.VMEM((_OUT,), jnp.int32),
        pltpu.VMEM((_OUT,), jnp.int32),
        pltpu.VMEM((_COUNT,), jnp.int32),
        pltpu.VMEM((4 * _OUT,), jnp.float32),
        pltpu.VMEM((4 * _COUNT,), jnp.float32),
        pltpu.VMEM((_RCH * _ROW,), jnp.float32),
        pltpu.VMEM((_RCH * _ROW,), jnp.float32),
        pltpu.VMEM((_RCH, _COUNT), jnp.float32),
        pltpu.VMEM((_RCH, _COUNT), jnp.float32),
        pltpu.SemaphoreType.DMA,
        pltpu.SemaphoreType.DMA,
        pltpu.SemaphoreType.DMA,
        pltpu.SemaphoreType.DMA,
    ],
)(_sc_body)


@jax.jit
def kernel(x, logits, code_logits, left_indices, right_indices, code_indices):
    co, dc = _coefs(logits, code_logits)
    return _sc_call(
        x.reshape(-1),
        left_indices.astype(jnp.int32),
        right_indices.astype(jnp.int32),
        code_indices.astype(jnp.int32),
        co.reshape(-1),
        dc.reshape(-1),
    )


# speedup vs baseline: 1.8394x; 1.0502x over previous
import functools

import jax
import jax.numpy as jnp
from jax import lax
from jax.experimental import pallas as pl
from jax.experimental.pallas import tpu as pltpu
from jax.experimental.pallas import tpu_sc as plsc

_B = 4096
_D = 2048
_C = 64
_OUT = 4096
_COUNT = 2048

_NC = 2
_NS = 16
_NW = _NC * _NS
_ROWS_PER_TILE = _B // _NW
_RCH = 8
_CHUNKS = _ROWS_PER_TILE // _RCH
_L = 16


def _coef_body(lg_ref, clg_ref, co_ref, dc_ref):
    w = 0.5 + 0.5 * jnp.sin(lg_ref[...])
    cw = 0.5 + 0.5 * jnp.sin(clg_ref[...])

    def poly(t):
        w00 = t[0:1, :]
        w01 = t[1:2, :]
        w10 = t[2:3, :]
        w11 = t[3:4, :]
        return jnp.concatenate(
            [w00, w10 - w00, w01 - w00, w11 - w10 - w01 + w00], axis=0)

    co = poly(w)
    cc = poly(cw)
    co_ref[...] = co
    dc_ref[...] = cc - co[:, :_COUNT]


def _coefs(logits, code_logits):
    return pl.pallas_call(
        _coef_body,
        out_shape=[
            jax.ShapeDtypeStruct((4, _OUT), jnp.float32),
            jax.ShapeDtypeStruct((4, _COUNT), jnp.float32),
        ],
    )(logits.T, code_logits.T)


_ROW = _D + _C


def _sc_body(x_hbm, lidx_hbm, ridx_hbm, cidx_hbm, co_hbm, dc_hbm, out_hbm,
             lidx_v, ridx_v, cidx_v, co_v, dc_v, xch0, xch1, ocha, ochb,
             isem0, isem1, osema, osemb):
    wid = lax.axis_index("s") * _NC + lax.axis_index("c")

    pltpu.sync_copy(lidx_hbm, lidx_v)
    pltpu.sync_copy(ridx_hbm, ridx_v)
    pltpu.sync_copy(cidx_hbm, cidx_v)
    pltpu.sync_copy(co_hbm, co_v)
    pltpu.sync_copy(dc_hbm, dc_v)

    row0 = wid * _ROWS_PER_TILE
    xchs = (xch0, xch1)
    isems = (isem0, isem1)

    def in_slice(ch):
        return x_hbm.at[pl.ds((row0 + ch * _RCH) * _ROW, _RCH * _ROW)]

    def outa_slice(ch):
        return out_hbm.at[pl.ds(row0 + ch * _RCH, _RCH), pl.ds(0, _COUNT)]

    def outb_slice(ch):
        return out_hbm.at[pl.ds(row0 + ch * _RCH, _RCH), pl.ds(_COUNT, _COUNT)]

    def compute_blended(xch, och, i0, ni):
        @plsc.parallel_loop(0, _COUNT // _L, unroll=1)
        def blended(jv):
            jo = jv * _L
            li = lidx_v[pl.ds(jo, _L)]
            ri = ridx_v[pl.ds(jo, _L)]
            ci = cidx_v[pl.ds(jo, _L)]
            a = co_v[pl.ds(jo, _L)]
            bl = co_v[pl.ds(_OUT + jo, _L)]
            br = co_v[pl.ds(2 * _OUT + jo, _L)]
            bb = co_v[pl.ds(3 * _OUT + jo, _L)]
            da = dc_v[pl.ds(jo, _L)]
            dbl = dc_v[pl.ds(_COUNT + jo, _L)]
            dbr = dc_v[pl.ds(2 * _COUNT + jo, _L)]
            dbb = dc_v[pl.ds(3 * _COUNT + jo, _L)]
            for i in range(i0, i0 + ni):
                l = plsc.load_gather(xch, [li + i * _ROW])
                r = plsc.load_gather(xch, [ri + i * _ROW])
                s = plsc.load_gather(xch, [ci + (i * _ROW + _D)])
                lr = l * r
                p = a + bl * l + br * r + bb * lr
                dp = da + dbl * l + dbr * r + dbb * lr
                och[i, pl.ds(jo, _L)] = p + s * dp

        del blended

    def compute_plain(xch, och, i0, ni):
        @plsc.parallel_loop(_COUNT // _L, _OUT // _L, unroll=1)
        def plain(jv):
            jo = jv * _L
            li = lidx_v[pl.ds(jo, _L)]
            ri = ridx_v[pl.ds(jo, _L)]
            a = co_v[pl.ds(jo, _L)]
            bl = co_v[pl.ds(_OUT + jo, _L)]
            br = co_v[pl.ds(2 * _OUT + jo, _L)]
            bb = co_v[pl.ds(3 * _OUT + jo, _L)]
            for i in range(i0, i0 + ni):
                l = plsc.load_gather(xch, [li + i * _ROW])
                r = plsc.load_gather(xch, [ri + i * _ROW])
                och[i, pl.ds(jo - _COUNT, _L)] = a + bl * l + br * r + bb * (l * r)

        del plain

    pltpu.async_copy(in_slice(0), xch0, isem0)

    def pair(p, carry):
        for b in range(2):
            ch = 2 * p + b

            @pl.when(ch + 1 < _CHUNKS)
            def _():
                pltpu.async_copy(in_slice(ch + 1), xchs[1 - b], isems[1 - b])

            pltpu.make_async_copy(in_slice(ch), xchs[b], isems[b]).wait()

            @pl.when(ch >= 1)
            def _():
                pltpu.make_async_copy(ocha, outa_slice(ch), osema).wait()

            compute_blended(xchs[b], ocha, 0, 4)
            compute_blended(xchs[b], ocha, 4, 4)
            pltpu.async_copy(ocha, outa_slice(ch), osema)

            @pl.when(ch >= 1)
            def _():
                pltpu.make_async_copy(ochb, outb_slice(ch), osemb).wait()

            compute_plain(xchs[b], ochb, 0, 4)
            compute_plain(xchs[b], ochb, 4, 4)
            pltpu.async_copy(ochb, outb_slice(ch), osemb)
        return carry

    lax.fori_loop(0, _CHUNKS // 2, pair, 0)

    pltpu.make_async_copy(ocha, outa_slice(_CHUNKS - 1), osema).wait()
    pltpu.make_async_copy(ochb, outb_slice(_CHUNKS - 1), osemb).wait()


_sc_call = functools.partial(
    pl.kernel,
    out_type=jax.ShapeDtypeStruct((_B, _OUT), jnp.float32),
    mesh=plsc.VectorSubcoreMesh(core_axis_name="c", subcore_axis_name="s"),
    compiler_params=pltpu.CompilerParams(needs_layout_passes=False),
    scratch_types=[
        pltpu.VMEM((_OUT,), jnp.int32),
        pltpu.VMEM((_OUT,), jnp.int32),
        pltpu.VMEM((_COUNT,), jnp.int32),
        pltpu.VMEM((4 * _OUT,), jnp.float32),
        pltpu.VMEM((4 * _COUNT,), jnp.float32),
        pltpu.VMEM((_RCH * _ROW,), jnp.float32),
        pltpu.VMEM((_RCH * _ROW,), jnp.float32),
        pltpu.VMEM((_RCH, _COUNT), jnp.float32),
        pltpu.VMEM((_RCH, _COUNT), jnp.float32),
        pltpu.SemaphoreType.DMA,
        pltpu.SemaphoreType.DMA,
        pltpu.SemaphoreType.DMA,
        pltpu.SemaphoreType.DMA,
    ],
)(_sc_body)


@jax.jit
def kernel(x, logits, code_logits, left_indices, right_indices, code_indices):
    co, dc = _coefs(logits, code_logits)
    return _sc_call(
        x.reshape(-1),
        left_indices.astype(jnp.int32),
        right_indices.astype(jnp.int32),
        code_indices.astype(jnp.int32),
        co.reshape(-1),
        dc.reshape(-1),
    )
